# Initial kernel scaffold; baseline (speedup 1.0000x reference)
import jax
import jax.numpy as jnp
from jax.experimental import pallas as pl


def kernel(x, W0, W1):
    raise NotImplementedError("write your pallas kernel here")


# speedup vs baseline: 1.8588x; 1.8588x over previous
import functools

import jax
import jax.numpy as jnp
from jax import lax
from jax.experimental import pallas as pl
from jax.experimental.pallas import tpu as pltpu
from jax.experimental.pallas import tpu_sc as plsc

NC = 2
NS = 16
NW = NC * NS
B_TOT = 4096 * 200
PER_W = B_TOT // NW
CH = 128
N_CH = PER_W // CH

_mesh = plsc.VectorSubcoreMesh(
    core_axis_name="c", subcore_axis_name="s", num_cores=NC, num_subcores=NS
)


@functools.partial(
    pl.kernel,
    out_type=jax.ShapeDtypeStruct((B_TOT, 2, 32), jnp.float32),
    mesh=_mesh,
    scratch_types=[
        pltpu.VMEM((PER_W,), jnp.int32),
        pltpu.VMEM((2, 2, CH, 32), jnp.float32),
        pltpu.SemaphoreType.DMA,
        pltpu.SemaphoreType.DMA,
    ],
    compiler_params=pltpu.CompilerParams(use_tc_tiling_on_sc=False),
)
def _emb_lookup(x_hbm, w0_hbm, w1_hbm, out_hbm, idx_v, rows, sem0, sem1):
    wid = lax.axis_index("s") * NC + lax.axis_index("c")
    base = wid * PER_W
    pltpu.sync_copy(x_hbm.at[pl.ds(base, PER_W)], idx_v)

    def gather_descs(c, slot):
        idx = idx_v.at[pl.ds(c * CH, CH)]
        return (
            pltpu.make_async_copy(w0_hbm.at[idx], rows.at[slot, 0], sem0),
            pltpu.make_async_copy(w1_hbm.at[idx], rows.at[slot, 1], sem1),
        )

    def gathers(c, slot):
        for d in gather_descs(c, slot):
            d.start()

    gathers(0, 0)

    @pl.loop(0, N_CH, step=2)
    def _(j):
        for k in range(2):
            c = j + k
            slot = k

            @pl.when(c + 1 < N_CH)
            def _():
                gathers(c + 1, 1 - slot)

            d0, d1 = gather_descs(c, slot)
            d0.wait()
            d1.wait()
            dst = out_hbm.at[pl.ds(base + c * CH, CH)]
            pltpu.sync_copy(rows.at[slot, 0], dst.at[:, 0])
            pltpu.sync_copy(rows.at[slot, 1], dst.at[:, 1])


def kernel(x, W0, W1):
    xf = x.reshape(-1).astype(jnp.int32)
    out = _emb_lookup(xf, W0, W1)
    return out.reshape(x.shape[0], x.shape[1], 64)


# speedup vs baseline: 1.8856x; 1.0144x over previous
import functools

import jax
import jax.numpy as jnp
from jax import lax
from jax.experimental import pallas as pl
from jax.experimental.pallas import tpu as pltpu
from jax.experimental.pallas import tpu_sc as plsc

NC = 2
NS = 16
NW = NC * NS
B_TOT = 4096 * 200
PER_W = B_TOT // NW
CH = 256
N_CH = PER_W // CH
NBUF = 4
F = 2

_mesh = plsc.VectorSubcoreMesh(
    core_axis_name="c", subcore_axis_name="s", num_cores=NC, num_subcores=NS
)


@functools.partial(
    pl.kernel,
    out_type=jax.ShapeDtypeStruct((B_TOT, 2, 32), jnp.float32),
    mesh=_mesh,
    scratch_types=[
        pltpu.VMEM((PER_W,), jnp.int32),
        pltpu.VMEM((NBUF, 2, CH, 32), jnp.float32),
        [pltpu.SemaphoreType.DMA] * NBUF,
        [pltpu.SemaphoreType.DMA] * NBUF,
    ],
    compiler_params=pltpu.CompilerParams(use_tc_tiling_on_sc=False),
)
def _emb_lookup(x_hbm, w0_hbm, w1_hbm, out_hbm, idx_v, rows, gsems, wsems):
    wid = lax.axis_index("s") * NC + lax.axis_index("c")
    base = wid * PER_W
    pltpu.sync_copy(x_hbm.at[pl.ds(base, PER_W)], idx_v)

    def gather_descs(c, slot):
        idx = idx_v.at[pl.ds(c * CH, CH)]
        return (
            pltpu.make_async_copy(w0_hbm.at[idx], rows.at[slot, 0], gsems[slot]),
            pltpu.make_async_copy(w1_hbm.at[idx], rows.at[slot, 1], gsems[slot]),
        )

    def write_descs(c, slot):
        dst = out_hbm.at[pl.ds(base + c * CH, CH)]
        return (
            pltpu.make_async_copy(rows.at[slot, 0], dst.at[:, 0], wsems[slot]),
            pltpu.make_async_copy(rows.at[slot, 1], dst.at[:, 1], wsems[slot]),
        )

    for c in range(F):
        for d in gather_descs(c, c % NBUF):
            d.start()

    @pl.loop(0, N_CH, step=NBUF)
    def _(j):
        for k in range(NBUF):
            c = j + k

            sf = (k + F) % NBUF

            @pl.when(c + F < N_CH)
            def _():
                @pl.when(c + F >= NBUF)
                def _():
                    for d in write_descs(c, sf):
                        d.wait()

                for d in gather_descs(c + F, sf):
                    d.start()

            for d in gather_descs(c, k):
                d.wait()
            for d in write_descs(c, k):
                d.start()

    for k in range(NBUF):
        for d in write_descs(0, k):
            d.wait()


def kernel(x, W0, W1):
    xf = x.reshape(-1).astype(jnp.int32)
    out = _emb_lookup(xf, W0, W1)
    return out.reshape(x.shape[0], x.shape[1], 64)


# speedup vs baseline: 1.9273x; 1.0221x over previous
import functools

import jax
import jax.numpy as jnp
from jax import lax
from jax.experimental import pallas as pl
from jax.experimental.pallas import tpu as pltpu
from jax.experimental.pallas import tpu_sc as plsc

VB = 512
NI = 489
S_PACK = NI * VB
V4 = 4 * S_PACK


def _transpose_body(r0, r1, r2, r3, o_ref):
    o_ref[...] = jnp.concatenate(
        [r0[...].T, r1[...].T, r2[...].T, r3[...].T], axis=1
    )


_LAST_BLK = 1953


def _relayout(wt):
    def mk(m):
        return pl.BlockSpec(
            (32, VB), lambda i, m=m: (0, jnp.minimum(NI * m + i, _LAST_BLK))
        )

    return pl.pallas_call(
        _transpose_body,
        grid=(NI,),
        in_specs=[mk(0), mk(1), mk(2), mk(3)],
        out_specs=pl.BlockSpec((VB, 128), lambda i: (i, 0)),
        out_shape=jax.ShapeDtypeStruct((S_PACK, 128), jnp.float32),
    )(wt, wt, wt, wt)


NC = 2
NS = 16
NW = NC * NS
B_TOT = 4096 * 200
PER_W = B_TOT // NW
CH = 256
N_CH = PER_W // CH
NBUF = 4
F = 2

_mesh = plsc.VectorSubcoreMesh(
    core_axis_name="c", subcore_axis_name="s", num_cores=NC, num_subcores=NS
)


@functools.partial(
    pl.kernel,
    out_type=jax.ShapeDtypeStruct((B_TOT, 2, 32), jnp.float32),
    mesh=_mesh,
    scratch_types=[
        pltpu.VMEM((PER_W,), jnp.int32),
        pltpu.VMEM((NBUF, 2, CH, 32), jnp.float32),
        [pltpu.SemaphoreType.DMA] * NBUF,
        [pltpu.SemaphoreType.DMA] * NBUF,
    ],
    compiler_params=pltpu.CompilerParams(use_tc_tiling_on_sc=False),
)
def _emb_lookup(x_hbm, w0_hbm, w1_hbm, out_hbm, idx_v, rows, gsems, wsems):
    wid = lax.axis_index("s") * NC + lax.axis_index("c")
    base = wid * PER_W
    pltpu.sync_copy(x_hbm.at[pl.ds(base, PER_W)], idx_v)

    def gather_descs(c, slot):
        idx = idx_v.at[pl.ds(c * CH, CH)]
        return (
            pltpu.make_async_copy(w0_hbm.at[idx], rows.at[slot, 0], gsems[slot]),
            pltpu.make_async_copy(w1_hbm.at[idx], rows.at[slot, 1], gsems[slot]),
        )

    def write_descs(c, slot):
        dst = out_hbm.at[pl.ds(base + c * CH, CH)]
        return (
            pltpu.make_async_copy(rows.at[slot, 0], dst.at[:, 0], wsems[slot]),
            pltpu.make_async_copy(rows.at[slot, 1], dst.at[:, 1], wsems[slot]),
        )

    for c in range(F):
        for d in gather_descs(c, c % NBUF):
            d.start()

    @pl.loop(0, N_CH, step=NBUF)
    def _(j):
        for k in range(NBUF):
            c = j + k

            sf = (k + F) % NBUF

            @pl.when(c + F < N_CH)
            def _():
                @pl.when(c + F >= NBUF)
                def _():
                    for d in write_descs(c, sf):
                        d.wait()

                for d in gather_descs(c + F, sf):
                    d.start()

            for d in gather_descs(c, k):
                d.wait()
            for d in write_descs(c, k):
                d.start()

    for k in range(NBUF):
        for d in write_descs(0, k):
            d.wait()


def kernel(x, W0, W1):
    r0 = _relayout(W0.T).reshape(V4, 32)
    r1 = _relayout(W1.T).reshape(V4, 32)
    xf = x.reshape(-1).astype(jnp.int32)
    m = (
        (xf >= S_PACK).astype(jnp.int32)
        + (xf >= 2 * S_PACK).astype(jnp.int32)
        + (xf >= 3 * S_PACK).astype(jnp.int32)
    )
    xr = 4 * xf - (4 * S_PACK - 1) * m
    out = _emb_lookup(xr, r0, r1)
    return out.reshape(x.shape[0], x.shape[1], 64)


# speedup vs baseline: 2.1380x; 1.1093x over previous
import functools

import jax
import jax.numpy as jnp
from jax import lax
from jax.experimental import pallas as pl
from jax.experimental.pallas import tpu as pltpu
from jax.experimental.pallas import tpu_sc as plsc

VB = 512
NI = 489
S_PACK = NI * VB
V4 = 4 * S_PACK


def _transpose_body(r0, r1, r2, r3, o_ref):
    a = jnp.concatenate([r0[...], r1[...], r2[...], r3[...]], axis=0)
    o_ref[...] = jax.lax.dot_general(
        a,
        jnp.eye(128, dtype=jnp.float32),
        (((0,), (0,)), ((), ())),
        preferred_element_type=jnp.float32,
    )


_LAST_BLK = 1953


def _relayout(wt):
    def mk(m):
        return pl.BlockSpec(
            (32, VB), lambda i, m=m: (0, jnp.minimum(NI * m + i, _LAST_BLK))
        )

    return pl.pallas_call(
        _transpose_body,
        grid=(NI,),
        in_specs=[mk(0), mk(1), mk(2), mk(3)],
        out_specs=pl.BlockSpec((VB, 128), lambda i: (i, 0)),
        out_shape=jax.ShapeDtypeStruct((S_PACK, 128), jnp.float32),
    )(wt, wt, wt, wt)


NC = 2
NS = 16
NW = NC * NS
B_TOT = 4096 * 200
PER_W = B_TOT // NW
CH = 256
N_CH = PER_W // CH
NBUF = 4
F = 2

_mesh = plsc.VectorSubcoreMesh(
    core_axis_name="c", subcore_axis_name="s", num_cores=NC, num_subcores=NS
)


@functools.partial(
    pl.kernel,
    out_type=jax.ShapeDtypeStruct((B_TOT, 2, 32), jnp.float32),
    mesh=_mesh,
    scratch_types=[
        pltpu.VMEM((PER_W,), jnp.int32),
        pltpu.VMEM((NBUF, 2, CH, 32), jnp.float32),
        [pltpu.SemaphoreType.DMA] * NBUF,
        [pltpu.SemaphoreType.DMA] * NBUF,
    ],
    compiler_params=pltpu.CompilerParams(use_tc_tiling_on_sc=False),
)
def _emb_lookup(x_hbm, w0_hbm, w1_hbm, out_hbm, idx_v, rows, gsems, wsems):
    wid = lax.axis_index("s") * NC + lax.axis_index("c")
    base = wid * PER_W
    pltpu.sync_copy(x_hbm.at[pl.ds(base, PER_W)], idx_v)

    def gather_descs(c, slot):
        idx = idx_v.at[pl.ds(c * CH, CH)]
        return (
            pltpu.make_async_copy(w0_hbm.at[idx], rows.at[slot, 0], gsems[slot]),
            pltpu.make_async_copy(w1_hbm.at[idx], rows.at[slot, 1], gsems[slot]),
        )

    def write_descs(c, slot):
        dst = out_hbm.at[pl.ds(base + c * CH, CH)]
        return (
            pltpu.make_async_copy(rows.at[slot, 0], dst.at[:, 0], wsems[slot]),
            pltpu.make_async_copy(rows.at[slot, 1], dst.at[:, 1], wsems[slot]),
        )

    for c in range(F):
        for d in gather_descs(c, c % NBUF):
            d.start()

    @pl.loop(0, N_CH, step=NBUF)
    def _(j):
        for k in range(NBUF):
            c = j + k

            sf = (k + F) % NBUF

            @pl.when(c + F < N_CH)
            def _():
                @pl.when(c + F >= NBUF)
                def _():
                    for d in write_descs(c, sf):
                        d.wait()

                for d in gather_descs(c + F, sf):
                    d.start()

            for d in gather_descs(c, k):
                d.wait()
            for d in write_descs(c, k):
                d.start()

    for k in range(NBUF):
        for d in write_descs(0, k):
            d.wait()


def kernel(x, W0, W1):
    r0 = _relayout(W0.T).reshape(V4, 32)
    r1 = _relayout(W1.T).reshape(V4, 32)
    xf = x.reshape(-1).astype(jnp.int32)
    m = (
        (xf >= S_PACK).astype(jnp.int32)
        + (xf >= 2 * S_PACK).astype(jnp.int32)
        + (xf >= 3 * S_PACK).astype(jnp.int32)
    )
    xr = 4 * xf - (4 * S_PACK - 1) * m
    out = _emb_lookup(xr, r0, r1)
    return out.reshape(x.shape[0], x.shape[1], 64)


# speedup vs baseline: 3.0466x; 1.4250x over previous
import functools

import jax
import jax.numpy as jnp
from jax import lax
from jax.experimental import pallas as pl
from jax.experimental.pallas import tpu as pltpu
from jax.experimental.pallas import tpu_sc as plsc

VB = 2048
NI = 123
S_PACK = NI * VB
V4 = 4 * S_PACK


def _transpose_body(r0, r1, r2, r3, o_ref):
    a = jnp.concatenate([r0[...], r1[...], r2[...], r3[...]], axis=0)
    o_ref[...] = jax.lax.dot_general(
        a,
        jnp.eye(128, dtype=jnp.float32),
        (((0,), (0,)), ((), ())),
        preferred_element_type=jnp.float32,
    )


_LAST_BLK = 488


def _relayout(wt):
    def mk(m):
        return pl.BlockSpec(
            (32, VB), lambda i, m=m: (0, jnp.minimum(NI * m + i, _LAST_BLK))
        )

    return pl.pallas_call(
        _transpose_body,
        grid=(NI,),
        in_specs=[mk(0), mk(1), mk(2), mk(3)],
        out_specs=pl.BlockSpec((VB, 128), lambda i: (i, 0)),
        out_shape=jax.ShapeDtypeStruct((S_PACK, 128), jnp.float32),
    )(wt, wt, wt, wt)


NC = 2
NS = 16
NW = NC * NS
B_TOT = 4096 * 200
PER_W = B_TOT // NW
CH = 256
N_CH = PER_W // CH
NBUF = 4
F = 2

_mesh = plsc.VectorSubcoreMesh(
    core_axis_name="c", subcore_axis_name="s", num_cores=NC, num_subcores=NS
)


@functools.partial(
    pl.kernel,
    out_type=jax.ShapeDtypeStruct((B_TOT, 2, 32), jnp.float32),
    mesh=_mesh,
    scratch_types=[
        pltpu.VMEM((PER_W,), jnp.int32),
        pltpu.VMEM((NBUF, 2, CH, 32), jnp.float32),
        [pltpu.SemaphoreType.DMA] * NBUF,
        [pltpu.SemaphoreType.DMA] * NBUF,
    ],
    compiler_params=pltpu.CompilerParams(use_tc_tiling_on_sc=False),
)
def _emb_lookup(x_hbm, w0_hbm, w1_hbm, out_hbm, idx_v, rows, gsems, wsems):
    wid = lax.axis_index("s") * NC + lax.axis_index("c")
    base = wid * PER_W
    pltpu.sync_copy(x_hbm.at[pl.ds(base, PER_W)], idx_v)

    def gather_descs(c, slot):
        idx = idx_v.at[pl.ds(c * CH, CH)]
        return (
            pltpu.make_async_copy(w0_hbm.at[idx], rows.at[slot, 0], gsems[slot]),
            pltpu.make_async_copy(w1_hbm.at[idx], rows.at[slot, 1], gsems[slot]),
        )

    def write_descs(c, slot):
        dst = out_hbm.at[pl.ds(base + c * CH, CH)]
        return (
            pltpu.make_async_copy(rows.at[slot, 0], dst.at[:, 0], wsems[slot]),
            pltpu.make_async_copy(rows.at[slot, 1], dst.at[:, 1], wsems[slot]),
        )

    for c in range(F):
        for d in gather_descs(c, c % NBUF):
            d.start()

    @pl.loop(0, N_CH, step=NBUF)
    def _(j):
        for k in range(NBUF):
            c = j + k

            sf = (k + F) % NBUF

            @pl.when(c + F < N_CH)
            def _():
                @pl.when(c + F >= NBUF)
                def _():
                    for d in write_descs(c, sf):
                        d.wait()

                for d in gather_descs(c + F, sf):
                    d.start()

            for d in gather_descs(c, k):
                d.wait()
            for d in write_descs(c, k):
                d.start()

    for k in range(NBUF):
        for d in write_descs(0, k):
            d.wait()


def kernel(x, W0, W1):
    r0 = _relayout(W0.T).reshape(V4, 32)
    r1 = _relayout(W1.T).reshape(V4, 32)
    xf = x.reshape(-1).astype(jnp.int32)
    m = (
        (xf >= S_PACK).astype(jnp.int32)
        + (xf >= 2 * S_PACK).astype(jnp.int32)
        + (xf >= 3 * S_PACK).astype(jnp.int32)
    )
    xr = 4 * xf - (4 * S_PACK - 1) * m
    out = _emb_lookup(xr, r0, r1)
    return out.reshape(x.shape[0], x.shape[1], 64)


# speedup vs baseline: 3.3272x; 1.0921x over previous
import functools

import jax
import jax.numpy as jnp
from jax import lax
from jax.experimental import pallas as pl
from jax.experimental.pallas import tpu as pltpu
from jax.experimental.pallas import tpu_sc as plsc

VB = 4096
NI = 62
S_PACK = NI * VB
V4 = 4 * S_PACK
_LAST_BLK = 244


def _transpose_body(r0, r1, r2, r3, o_ref):
    a = jnp.concatenate([r0[...], r1[...], r2[...], r3[...]], axis=0)
    o_ref[...] = jax.lax.dot_general(
        a,
        jnp.eye(128, dtype=jnp.float32),
        (((0,), (0,)), ((), ())),
        preferred_element_type=jnp.float32,
    )


def _relayout(wt):
    def mk(m):
        return pl.BlockSpec(
            (32, VB), lambda i, m=m: (0, jnp.minimum(NI * m + i, _LAST_BLK))
        )

    return pl.pallas_call(
        _transpose_body,
        grid=(NI,),
        in_specs=[mk(0), mk(1), mk(2), mk(3)],
        out_specs=pl.BlockSpec((VB, 128), lambda i: (i, 0)),
        out_shape=jax.ShapeDtypeStruct((S_PACK, 128), jnp.float32),
    )(wt, wt, wt, wt)


NC = 2
NS = 16
NW = NC * NS
B_TOT = 4096 * 200
PER_W = B_TOT // NW
CH = 320
N_CH = PER_W // CH
NBUF = 4
F = 2

_mesh = plsc.VectorSubcoreMesh(
    core_axis_name="c", subcore_axis_name="s", num_cores=NC, num_subcores=NS
)


@functools.partial(
    pl.kernel,
    out_type=jax.ShapeDtypeStruct((B_TOT, 2, 32), jnp.float32),
    mesh=_mesh,
    scratch_types=[
        pltpu.VMEM((PER_W,), jnp.int32),
        pltpu.VMEM((NBUF, 2, CH, 32), jnp.float32),
        [pltpu.SemaphoreType.DMA] * NBUF,
        [pltpu.SemaphoreType.DMA] * NBUF,
    ],
    compiler_params=pltpu.CompilerParams(use_tc_tiling_on_sc=False),
)
def _emb_lookup(x_hbm, w0_hbm, w1_hbm, out_hbm, idx_v, rows, gsems, wsems):
    wid = lax.axis_index("s") * NC + lax.axis_index("c")
    base = wid * PER_W
    pltpu.sync_copy(x_hbm.at[pl.ds(base, PER_W)], idx_v)

    def gather_descs(c, slot):
        idx = idx_v.at[pl.ds(c * CH, CH)]
        return (
            pltpu.make_async_copy(w0_hbm.at[idx], rows.at[slot, 0], gsems[slot]),
            pltpu.make_async_copy(w1_hbm.at[idx], rows.at[slot, 1], gsems[slot]),
        )

    def write_descs(c, slot):
        dst = out_hbm.at[pl.ds(base + c * CH, CH)]
        return (
            pltpu.make_async_copy(rows.at[slot, 0], dst.at[:, 0], wsems[slot]),
            pltpu.make_async_copy(rows.at[slot, 1], dst.at[:, 1], wsems[slot]),
        )

    for c in range(F):
        for d in gather_descs(c, c % NBUF):
            d.start()

    @pl.loop(0, N_CH, step=NBUF)
    def _(j):
        for k in range(NBUF):
            c = j + k

            sf = (k + F) % NBUF

            @pl.when(c + F < N_CH)
            def _():
                @pl.when(c + F >= NBUF)
                def _():
                    for d in write_descs(c, sf):
                        d.wait()

                for d in gather_descs(c + F, sf):
                    d.start()

            for d in gather_descs(c, k):
                d.wait()
            for d in write_descs(c, k):
                d.start()

    for k in range(NBUF):
        for d in write_descs(0, k):
            d.wait()


def kernel(x, W0, W1):
    r0 = _relayout(W0.T).reshape(V4, 32)
    r1 = _relayout(W1.T).reshape(V4, 32)
    xf = x.reshape(-1).astype(jnp.int32)
    m = (
        (xf >= S_PACK).astype(jnp.int32)
        + (xf >= 2 * S_PACK).astype(jnp.int32)
        + (xf >= 3 * S_PACK).astype(jnp.int32)
    )
    xr = 4 * xf - (4 * S_PACK - 1) * m
    out = _emb_lookup(xr, r0, r1)
    return out.reshape(x.shape[0], x.shape[1], 64)


# speedup vs baseline: 3.4446x; 1.0353x over previous
import functools

import jax
import jax.numpy as jnp
from jax import lax
from jax.experimental import pallas as pl
from jax.experimental.pallas import tpu as pltpu
from jax.experimental.pallas import tpu_sc as plsc

VB = 8192
NI = 31
S_PACK = NI * VB
V4 = 4 * S_PACK
_LAST_BLK = 122


def _transpose_body(r0, r1, r2, r3, o_ref):
    a = jnp.concatenate([r0[...], r1[...], r2[...], r3[...]], axis=0)
    o_ref[...] = jax.lax.dot_general(
        a,
        jnp.eye(128, dtype=jnp.float32),
        (((0,), (0,)), ((), ())),
        preferred_element_type=jnp.float32,
    )


def _relayout(wt):
    def mk(m):
        return pl.BlockSpec(
            (32, VB), lambda i, m=m: (0, jnp.minimum(NI * m + i, _LAST_BLK))
        )

    return pl.pallas_call(
        _transpose_body,
        grid=(NI,),
        in_specs=[mk(0), mk(1), mk(2), mk(3)],
        out_specs=pl.BlockSpec((VB, 128), lambda i: (i, 0)),
        out_shape=jax.ShapeDtypeStruct((S_PACK, 128), jnp.float32),
    )(wt, wt, wt, wt)


NC = 2
NS = 16
NW = NC * NS
B_TOT = 4096 * 200
PER_W = B_TOT // NW
CH = 320
N_CH = PER_W // CH
NBUF = 4
F = 2

_mesh = plsc.VectorSubcoreMesh(
    core_axis_name="c", subcore_axis_name="s", num_cores=NC, num_subcores=NS
)


@functools.partial(
    pl.kernel,
    out_type=jax.ShapeDtypeStruct((B_TOT, 2, 32), jnp.float32),
    mesh=_mesh,
    scratch_types=[
        pltpu.VMEM((PER_W,), jnp.int32),
        pltpu.VMEM((NBUF, 2, CH, 32), jnp.float32),
        [pltpu.SemaphoreType.DMA] * NBUF,
        [pltpu.SemaphoreType.DMA] * NBUF,
    ],
    compiler_params=pltpu.CompilerParams(use_tc_tiling_on_sc=False),
)
def _emb_lookup(x_hbm, w0_hbm, w1_hbm, out_hbm, idx_v, rows, gsems, wsems):
    wid = lax.axis_index("s") * NC + lax.axis_index("c")
    base = wid * PER_W
    pltpu.sync_copy(x_hbm.at[pl.ds(base, PER_W)], idx_v)

    def gather_descs(c, slot):
        idx = idx_v.at[pl.ds(c * CH, CH)]
        return (
            pltpu.make_async_copy(w0_hbm.at[idx], rows.at[slot, 0], gsems[slot]),
            pltpu.make_async_copy(w1_hbm.at[idx], rows.at[slot, 1], gsems[slot]),
        )

    def write_descs(c, slot):
        dst = out_hbm.at[pl.ds(base + c * CH, CH)]
        return (
            pltpu.make_async_copy(rows.at[slot, 0], dst.at[:, 0], wsems[slot]),
            pltpu.make_async_copy(rows.at[slot, 1], dst.at[:, 1], wsems[slot]),
        )

    for c in range(F):
        for d in gather_descs(c, c % NBUF):
            d.start()

    @pl.loop(0, N_CH, step=NBUF)
    def _(j):
        for k in range(NBUF):
            c = j + k

            sf = (k + F) % NBUF

            @pl.when(c + F < N_CH)
            def _():
                @pl.when(c + F >= NBUF)
                def _():
                    for d in write_descs(c, sf):
                        d.wait()

                for d in gather_descs(c + F, sf):
                    d.start()

            for d in gather_descs(c, k):
                d.wait()
            for d in write_descs(c, k):
                d.start()

    for k in range(NBUF):
        for d in write_descs(0, k):
            d.wait()


def kernel(x, W0, W1):
    r0 = _relayout(W0.T).reshape(V4, 32)
    r1 = _relayout(W1.T).reshape(V4, 32)
    xf = x.reshape(-1).astype(jnp.int32)
    m = (
        (xf >= S_PACK).astype(jnp.int32)
        + (xf >= 2 * S_PACK).astype(jnp.int32)
        + (xf >= 3 * S_PACK).astype(jnp.int32)
    )
    xr = 4 * xf - (4 * S_PACK - 1) * m
    out = _emb_lookup(xr, r0, r1)
    return out.reshape(x.shape[0], x.shape[1], 64)
